# baseline (speedup 1.0000x reference)
import functools

import jax
import jax.numpy as jnp
from jax import lax
from jax.experimental import pallas as pl
from jax.experimental.pallas import tpu as pltpu
from jax.experimental.pallas import tpu_sc as plsc

_B = 16384
_F = 64
_NC = 2
_NS = 16
_NW = _NC * _NS
_SPLIT = 8192
_TCN = _B - _SPLIT
_BPW = _SPLIT // _NW
_GROUPS = _BPW // 16
_NBUF = 4


def _body(uc_ref, ic_ref, ufT_ref, ifT_ref, wb_ref,
          out_ref,
          idx_u, idx_i, u_tile, i_tile, wb_v, out_v, sems):
    wid = lax.axis_index("s") * _NC + lax.axis_index("c")
    base = wid * _BPW

    pltpu.sync_copy(uc_ref.at[pl.ds(base, _BPW)], idx_u)
    pltpu.sync_copy(ic_ref.at[pl.ds(base, _BPW)], idx_i)
    pltpu.sync_copy(wb_ref, wb_v)

    w = [wb_v[pl.ds(c * 16, 16)] for c in range(4)]
    b = wb_v[pl.ds(64, 16)][0]
    lane = lax.iota(jnp.int32, 16)

    def fetch(ru, ri, slot):
        pltpu.make_async_copy(
            ufT_ref.at[:, pl.ds((ru // 128) * 128, 128)],
            u_tile.at[slot], sems.at[slot]).start()
        pltpu.make_async_copy(
            ifT_ref.at[:, pl.ds((ri // 128) * 128, 128)],
            i_tile.at[slot], sems.at[slot]).start()

    def wait_slot(slot):
        pltpu.make_async_copy(
            ufT_ref.at[:, pl.ds(0, 128)], u_tile.at[slot],
            sems.at[slot]).wait()
        pltpu.make_async_copy(
            ifT_ref.at[:, pl.ds(0, 128)], i_tile.at[slot],
            sems.at[slot]).wait()

    ru0 = idx_u[pl.ds(0, 16)]
    ri0 = idx_i[pl.ds(0, 16)]
    for s in range(_NBUF):
        fetch(ru0[s], ri0[s], s)

    def group(g, carry):
        j0 = g * 16
        ru_vec = idx_u[pl.ds(j0, 16)]
        ri_vec = idx_i[pl.ds(j0, 16)]
        nxt = jnp.minimum(j0 + 16, _BPW - 16)
        ru_nxt = idx_u[pl.ds(nxt, 16)]
        ri_nxt = idx_i[pl.ds(nxt, 16)]
        last = g == _GROUPS - 1
        res = jnp.zeros((16,), jnp.float32)
        for jj in range(16):
            slot = jj % _NBUF
            ru = ru_vec[jj]
            ri = ri_vec[jj]
            wait_slot(slot)
            cuv = jnp.full((16,), ru % 128, jnp.int32)
            civ = jnp.full((16,), ri % 128, jnp.int32)
            acc = jnp.zeros((16,), jnp.float32)
            for c in range(4):
                fl = c * 16 + lane
                uv = plsc.load_gather(u_tile.at[slot], [fl, cuv])
                iv = plsc.load_gather(i_tile.at[slot], [fl, civ])
                acc = acc + uv * iv * w[c]
            if jj < 16 - _NBUF:
                run, rin = ru_vec[jj + _NBUF], ri_vec[jj + _NBUF]
            else:
                run = jnp.where(last, ru, ru_nxt[jj + _NBUF - 16])
                rin = jnp.where(last, ri, ri_nxt[jj + _NBUF - 16])
            fetch(run, rin, slot)
            s = jnp.sum(acc) + b
            res = jnp.where(lane == jj, jnp.maximum(s, 0.0), res)
        out_v[pl.ds(j0, 16)] = res
        return carry

    lax.fori_loop(0, _GROUPS, group, 0)

    for s in range(_NBUF):
        wait_slot(s)

    pltpu.sync_copy(out_v, out_ref.at[pl.ds(base, _BPW)])


def _tc_body(ucr, icr, u_blk, i_blk, w_ref, b_ref, out_ref):
    g = pl.program_id(0)
    cu = ucr[g] % 128
    ci = icr[g] % 128
    li = lax.broadcasted_iota(jnp.int32, (_F, 128), 1)
    colu = jnp.sum(jnp.where(li == cu, u_blk[...], 0.0), axis=1,
                   keepdims=True)
    coli = jnp.sum(jnp.where(li == ci, i_blk[...], 0.0), axis=1,
                   keepdims=True)
    s = jnp.maximum(jnp.sum(colu * coli * w_ref[...]) + b_ref[0, 0], 0.0)
    pos3 = (lax.broadcasted_iota(jnp.int32, (1, 8, 128), 1) * 128
            + lax.broadcasted_iota(jnp.int32, (1, 8, 128), 2))
    out_ref[...] = jnp.where(pos3 == g % 1024, s, out_ref[...])


def _tc_half(uc, ic, ufT, ifT, w2, b2):
    grid_spec = pltpu.PrefetchScalarGridSpec(
        num_scalar_prefetch=2,
        grid=(_TCN,),
        in_specs=[
            pl.BlockSpec((_F, 128), lambda g, ucr, icr: (0, ucr[g] // 128)),
            pl.BlockSpec((_F, 128), lambda g, ucr, icr: (0, icr[g] // 128)),
            pl.BlockSpec((_F, 1), lambda g, ucr, icr: (0, 0)),
            pl.BlockSpec((1, 1), lambda g, ucr, icr: (0, 0)),
        ],
        out_specs=pl.BlockSpec((1, 8, 128),
                               lambda g, ucr, icr: (g // 1024, 0, 0)),
    )
    out = pl.pallas_call(
        _tc_body,
        grid_spec=grid_spec,
        out_shape=jax.ShapeDtypeStruct((_TCN // 1024, 8, 128), jnp.float32),
    )(uc, ic, ufT, ifT, w2, b2)
    return out.reshape(_TCN)


@jax.jit
def kernel(user_coordinates, item_coordinates, USER_factors, ITEM_factors,
           W, b):
    uc = user_coordinates.astype(jnp.int32)
    ic = item_coordinates.astype(jnp.int32)
    ufT = USER_factors.T
    ifT = ITEM_factors.T
    wb = jnp.concatenate([W.reshape(_F), b.reshape(1),
                          jnp.zeros((15,), jnp.float32)])

    mesh = plsc.VectorSubcoreMesh(core_axis_name="c", subcore_axis_name="s")
    run = pl.kernel(
        _body,
        mesh=mesh,
        compiler_params=pltpu.CompilerParams(needs_layout_passes=False),
        out_type=jax.ShapeDtypeStruct((_SPLIT,), jnp.float32),
        scratch_types=[
            pltpu.VMEM((_BPW,), jnp.int32),
            pltpu.VMEM((_BPW,), jnp.int32),
            pltpu.VMEM((_NBUF, _F, 128), jnp.float32),
            pltpu.VMEM((_NBUF, _F, 128), jnp.float32),
            pltpu.VMEM((80,), jnp.float32),
            pltpu.VMEM((_BPW,), jnp.float32),
            pltpu.SemaphoreType.DMA((_NBUF,)),
        ],
    )
    sc_out = run(uc[:_SPLIT], ic[:_SPLIT], ufT, ifT, wb)
    tc_out = _tc_half(uc[_SPLIT:], ic[_SPLIT:], ufT, ifT,
                      W.reshape(_F, 1), b.reshape(1, 1))
    return jnp.concatenate([sc_out, tc_out]).reshape(_B, 1)


# speedup vs baseline: 4.6025x; 4.6025x over previous
import functools

import jax
import jax.numpy as jnp
from jax import lax
from jax.experimental import pallas as pl
from jax.experimental.pallas import tpu as pltpu
from jax.experimental.pallas import tpu_sc as plsc

_B = 16384
_F = 64
_NC = 2
_NS = 16
_NW = _NC * _NS
_SPLIT = 8192
_TCN = _B - _SPLIT
_RPS = 8
_BPW = _SPLIT // _NW
_GROUPS = _BPW // 16
_NBUF = 4


def _body(uc_ref, ic_ref, ufT_ref, ifT_ref, wb_ref,
          out_ref,
          idx_u, idx_i, u_tile, i_tile, wb_v, out_v, sems):
    wid = lax.axis_index("s") * _NC + lax.axis_index("c")
    base = wid * _BPW

    pltpu.sync_copy(uc_ref.at[pl.ds(base, _BPW)], idx_u)
    pltpu.sync_copy(ic_ref.at[pl.ds(base, _BPW)], idx_i)
    pltpu.sync_copy(wb_ref, wb_v)

    w = [wb_v[pl.ds(c * 16, 16)] for c in range(4)]
    b = wb_v[pl.ds(64, 16)][0]
    lane = lax.iota(jnp.int32, 16)

    def fetch(ru, ri, slot):
        pltpu.make_async_copy(
            ufT_ref.at[:, pl.ds((ru // 128) * 128, 128)],
            u_tile.at[slot], sems.at[slot]).start()
        pltpu.make_async_copy(
            ifT_ref.at[:, pl.ds((ri // 128) * 128, 128)],
            i_tile.at[slot], sems.at[slot]).start()

    def wait_slot(slot):
        pltpu.make_async_copy(
            ufT_ref.at[:, pl.ds(0, 128)], u_tile.at[slot],
            sems.at[slot]).wait()
        pltpu.make_async_copy(
            ifT_ref.at[:, pl.ds(0, 128)], i_tile.at[slot],
            sems.at[slot]).wait()

    ru0 = idx_u[pl.ds(0, 16)]
    ri0 = idx_i[pl.ds(0, 16)]
    for s in range(_NBUF):
        fetch(ru0[s], ri0[s], s)

    def group(g, carry):
        j0 = g * 16
        ru_vec = idx_u[pl.ds(j0, 16)]
        ri_vec = idx_i[pl.ds(j0, 16)]
        nxt = jnp.minimum(j0 + 16, _BPW - 16)
        ru_nxt = idx_u[pl.ds(nxt, 16)]
        ri_nxt = idx_i[pl.ds(nxt, 16)]
        last = g == _GROUPS - 1
        res = jnp.zeros((16,), jnp.float32)
        for jj in range(16):
            slot = jj % _NBUF
            ru = ru_vec[jj]
            ri = ri_vec[jj]
            wait_slot(slot)
            cuv = jnp.full((16,), ru % 128, jnp.int32)
            civ = jnp.full((16,), ri % 128, jnp.int32)
            acc = jnp.zeros((16,), jnp.float32)
            for c in range(4):
                fl = c * 16 + lane
                uv = plsc.load_gather(u_tile.at[slot], [fl, cuv])
                iv = plsc.load_gather(i_tile.at[slot], [fl, civ])
                acc = acc + uv * iv * w[c]
            if jj < 16 - _NBUF:
                run, rin = ru_vec[jj + _NBUF], ri_vec[jj + _NBUF]
            else:
                run = jnp.where(last, ru, ru_nxt[jj + _NBUF - 16])
                rin = jnp.where(last, ri, ri_nxt[jj + _NBUF - 16])
            fetch(run, rin, slot)
            s = jnp.sum(acc) + b
            res = jnp.where(lane == jj, jnp.maximum(s, 0.0), res)
        out_v[pl.ds(j0, 16)] = res
        return carry

    lax.fori_loop(0, _GROUPS, group, 0)

    for s in range(_NBUF):
        wait_slot(s)

    pltpu.sync_copy(out_v, out_ref.at[pl.ds(base, _BPW)])


def _tc_body(ucr, icr, *refs):
    blks = refs[:2 * _RPS]
    w_ref, b_ref, out_ref = refs[2 * _RPS:]
    g = pl.program_id(0)
    si = lax.broadcasted_iota(jnp.int32, (128, 1), 0)
    pos3 = (lax.broadcasted_iota(jnp.int32, (1, 8, 128), 1) * 128
            + lax.broadcasted_iota(jnp.int32, (1, 8, 128), 2))
    res = out_ref[...]
    for r in range(_RPS):
        cu = ucr[g * _RPS + r] % 128
        ci = icr[g * _RPS + r] % 128
        colu = jnp.dot(blks[r][...], (si == cu).astype(jnp.float32))
        coli = jnp.dot(blks[_RPS + r][...], (si == ci).astype(jnp.float32))
        s = jnp.maximum(jnp.sum(colu * coli * w_ref[...]) + b_ref[0, 0], 0.0)
        res = jnp.where(pos3 == (g % 128) * _RPS + r, s, res)
    out_ref[...] = res


def _tc_half(uc, ic, ufT, ifT, w2, b2):
    nsteps = _TCN // _RPS
    u_specs = [
        pl.BlockSpec((_F, 128),
                     lambda g, ucr, icr, r=r: (0, ucr[g * _RPS + r] // 128))
        for r in range(_RPS)
    ]
    i_specs = [
        pl.BlockSpec((_F, 128),
                     lambda g, ucr, icr, r=r: (0, icr[g * _RPS + r] // 128))
        for r in range(_RPS)
    ]
    grid_spec = pltpu.PrefetchScalarGridSpec(
        num_scalar_prefetch=2,
        grid=(nsteps,),
        in_specs=u_specs + i_specs + [
            pl.BlockSpec((_F, 1), lambda g, ucr, icr: (0, 0)),
            pl.BlockSpec((1, 1), lambda g, ucr, icr: (0, 0)),
        ],
        out_specs=pl.BlockSpec((1, 8, 128),
                               lambda g, ucr, icr: (g // 128, 0, 0)),
    )
    out = pl.pallas_call(
        _tc_body,
        grid_spec=grid_spec,
        out_shape=jax.ShapeDtypeStruct((_TCN // 1024, 8, 128), jnp.float32),
    )(uc, ic, *([ufT] * _RPS), *([ifT] * _RPS), w2, b2)
    return out.reshape(_TCN)


@jax.jit
def kernel(user_coordinates, item_coordinates, USER_factors, ITEM_factors,
           W, b):
    uc = user_coordinates.astype(jnp.int32)
    ic = item_coordinates.astype(jnp.int32)
    ufT = USER_factors.T
    ifT = ITEM_factors.T
    wb = jnp.concatenate([W.reshape(_F), b.reshape(1),
                          jnp.zeros((15,), jnp.float32)])

    mesh = plsc.VectorSubcoreMesh(core_axis_name="c", subcore_axis_name="s")
    run = pl.kernel(
        _body,
        mesh=mesh,
        compiler_params=pltpu.CompilerParams(needs_layout_passes=False),
        out_type=jax.ShapeDtypeStruct((_SPLIT,), jnp.float32),
        scratch_types=[
            pltpu.VMEM((_BPW,), jnp.int32),
            pltpu.VMEM((_BPW,), jnp.int32),
            pltpu.VMEM((_NBUF, _F, 128), jnp.float32),
            pltpu.VMEM((_NBUF, _F, 128), jnp.float32),
            pltpu.VMEM((80,), jnp.float32),
            pltpu.VMEM((_BPW,), jnp.float32),
            pltpu.SemaphoreType.DMA((_NBUF,)),
        ],
    )
    sc_out = run(uc[:_SPLIT], ic[:_SPLIT], ufT, ifT, wb)
    tc_out = _tc_half(uc[_SPLIT:], ic[_SPLIT:], ufT, ifT,
                      W.reshape(_F, 1), b.reshape(1, 1))
    return jnp.concatenate([sc_out, tc_out]).reshape(_B, 1)


# speedup vs baseline: 10.6542x; 2.3149x over previous
import functools

import jax
import jax.numpy as jnp
from jax import lax
from jax.experimental import pallas as pl
from jax.experimental.pallas import tpu as pltpu
from jax.experimental.pallas import tpu_sc as plsc

_B = 16384
_F = 64
_NC = 2
_NS = 16
_NW = _NC * _NS
_SPLIT = 13312
_TCN = _B - _SPLIT
_RPS = 8
_BPW = _SPLIT // _NW
_GROUPS = _BPW // 16
_NBUF = 4


def _body(uc_ref, ic_ref, ufT_ref, ifT_ref, wb_ref,
          out_ref,
          idx_u, idx_i, u_tile, i_tile, wb_v, out_v, sems):
    wid = lax.axis_index("s") * _NC + lax.axis_index("c")
    base = wid * _BPW

    pltpu.sync_copy(uc_ref.at[pl.ds(base, _BPW)], idx_u)
    pltpu.sync_copy(ic_ref.at[pl.ds(base, _BPW)], idx_i)
    pltpu.sync_copy(wb_ref, wb_v)

    w = [wb_v[pl.ds(c * 16, 16)] for c in range(4)]
    b = wb_v[pl.ds(64, 16)][0]
    lane = lax.iota(jnp.int32, 16)

    def fetch(ru, ri, slot):
        pltpu.make_async_copy(
            ufT_ref.at[:, pl.ds((ru // 128) * 128, 128)],
            u_tile.at[slot], sems.at[slot]).start()
        pltpu.make_async_copy(
            ifT_ref.at[:, pl.ds((ri // 128) * 128, 128)],
            i_tile.at[slot], sems.at[slot]).start()

    def wait_slot(slot):
        pltpu.make_async_copy(
            ufT_ref.at[:, pl.ds(0, 128)], u_tile.at[slot],
            sems.at[slot]).wait()
        pltpu.make_async_copy(
            ifT_ref.at[:, pl.ds(0, 128)], i_tile.at[slot],
            sems.at[slot]).wait()

    ru0 = idx_u[pl.ds(0, 16)]
    ri0 = idx_i[pl.ds(0, 16)]
    for s in range(_NBUF):
        fetch(ru0[s], ri0[s], s)

    def group(g, carry):
        j0 = g * 16
        ru_vec = idx_u[pl.ds(j0, 16)]
        ri_vec = idx_i[pl.ds(j0, 16)]
        nxt = jnp.minimum(j0 + 16, _BPW - 16)
        ru_nxt = idx_u[pl.ds(nxt, 16)]
        ri_nxt = idx_i[pl.ds(nxt, 16)]
        last = g == _GROUPS - 1
        res = jnp.zeros((16,), jnp.float32)
        for jj in range(16):
            slot = jj % _NBUF
            ru = ru_vec[jj]
            ri = ri_vec[jj]
            wait_slot(slot)
            cuv = jnp.full((16,), ru % 128, jnp.int32)
            civ = jnp.full((16,), ri % 128, jnp.int32)
            acc = jnp.zeros((16,), jnp.float32)
            for c in range(4):
                fl = c * 16 + lane
                uv = plsc.load_gather(u_tile.at[slot], [fl, cuv])
                iv = plsc.load_gather(i_tile.at[slot], [fl, civ])
                acc = acc + uv * iv * w[c]
            if jj < 16 - _NBUF:
                run, rin = ru_vec[jj + _NBUF], ri_vec[jj + _NBUF]
            else:
                run = jnp.where(last, ru, ru_nxt[jj + _NBUF - 16])
                rin = jnp.where(last, ri, ri_nxt[jj + _NBUF - 16])
            fetch(run, rin, slot)
            s = jnp.sum(acc) + b
            res = jnp.where(lane == jj, jnp.maximum(s, 0.0), res)
        out_v[pl.ds(j0, 16)] = res
        return carry

    lax.fori_loop(0, _GROUPS, group, 0)

    for s in range(_NBUF):
        wait_slot(s)

    pltpu.sync_copy(out_v, out_ref.at[pl.ds(base, _BPW)])


def _tc_body(ucr, icr, *refs):
    blks = refs[:2 * _RPS]
    w_ref, b_ref, out_ref = refs[2 * _RPS:]
    g = pl.program_id(0)
    si = lax.broadcasted_iota(jnp.int32, (128, 1), 0)
    pos3 = (lax.broadcasted_iota(jnp.int32, (1, 8, 128), 1) * 128
            + lax.broadcasted_iota(jnp.int32, (1, 8, 128), 2))
    res = out_ref[...]
    for r in range(_RPS):
        cu = ucr[g * _RPS + r] % 128
        ci = icr[g * _RPS + r] % 128
        colu = jnp.dot(blks[r][...], (si == cu).astype(jnp.float32))
        coli = jnp.dot(blks[_RPS + r][...], (si == ci).astype(jnp.float32))
        s = jnp.maximum(jnp.sum(colu * coli * w_ref[...]) + b_ref[0, 0], 0.0)
        res = jnp.where(pos3 == (g % 128) * _RPS + r, s, res)
    out_ref[...] = res


def _tc_half(uc, ic, ufT, ifT, w2, b2):
    nsteps = _TCN // _RPS
    u_specs = [
        pl.BlockSpec((_F, 128),
                     lambda g, ucr, icr, r=r: (0, ucr[g * _RPS + r] // 128))
        for r in range(_RPS)
    ]
    i_specs = [
        pl.BlockSpec((_F, 128),
                     lambda g, ucr, icr, r=r: (0, icr[g * _RPS + r] // 128))
        for r in range(_RPS)
    ]
    grid_spec = pltpu.PrefetchScalarGridSpec(
        num_scalar_prefetch=2,
        grid=(nsteps,),
        in_specs=u_specs + i_specs + [
            pl.BlockSpec((_F, 1), lambda g, ucr, icr: (0, 0)),
            pl.BlockSpec((1, 1), lambda g, ucr, icr: (0, 0)),
        ],
        out_specs=pl.BlockSpec((1, 8, 128),
                               lambda g, ucr, icr: (g // 128, 0, 0)),
    )
    out = pl.pallas_call(
        _tc_body,
        grid_spec=grid_spec,
        out_shape=jax.ShapeDtypeStruct((_TCN // 1024, 8, 128), jnp.float32),
    )(uc, ic, *([ufT] * _RPS), *([ifT] * _RPS), w2, b2)
    return out.reshape(_TCN)


@jax.jit
def kernel(user_coordinates, item_coordinates, USER_factors, ITEM_factors,
           W, b):
    uc = user_coordinates.astype(jnp.int32)
    ic = item_coordinates.astype(jnp.int32)
    ufT = USER_factors.T
    ifT = ITEM_factors.T
    wb = jnp.concatenate([W.reshape(_F), b.reshape(1),
                          jnp.zeros((15,), jnp.float32)])

    mesh = plsc.VectorSubcoreMesh(core_axis_name="c", subcore_axis_name="s")
    run = pl.kernel(
        _body,
        mesh=mesh,
        compiler_params=pltpu.CompilerParams(needs_layout_passes=False),
        out_type=jax.ShapeDtypeStruct((_SPLIT,), jnp.float32),
        scratch_types=[
            pltpu.VMEM((_BPW,), jnp.int32),
            pltpu.VMEM((_BPW,), jnp.int32),
            pltpu.VMEM((_NBUF, _F, 128), jnp.float32),
            pltpu.VMEM((_NBUF, _F, 128), jnp.float32),
            pltpu.VMEM((80,), jnp.float32),
            pltpu.VMEM((_BPW,), jnp.float32),
            pltpu.SemaphoreType.DMA((_NBUF,)),
        ],
    )
    sc_out = run(uc[:_SPLIT], ic[:_SPLIT], ufT, ifT, wb)
    tc_out = _tc_half(uc[_SPLIT:], ic[_SPLIT:], ufT, ifT,
                      W.reshape(_F, 1), b.reshape(1, 1))
    return jnp.concatenate([sc_out, tc_out]).reshape(_B, 1)


# speedup vs baseline: 12.4390x; 1.1675x over previous
import functools

import jax
import jax.numpy as jnp
from jax import lax
from jax.experimental import pallas as pl
from jax.experimental.pallas import tpu as pltpu
from jax.experimental.pallas import tpu_sc as plsc

_B = 16384
_F = 64
_NC = 2
_NS = 16
_NW = _NC * _NS
_SPLIT = 14336
_TCN = _B - _SPLIT
_RPS = 8
_BPW = _SPLIT // _NW
_GROUPS = _BPW // 16
_NBUF = 4


def _body(uc_ref, ic_ref, ufT_ref, ifT_ref, wb_ref,
          out_ref,
          idx_u, idx_i, u_tile, i_tile, wb_v, out_v, sems):
    wid = lax.axis_index("s") * _NC + lax.axis_index("c")
    base = wid * _BPW

    pltpu.sync_copy(uc_ref.at[pl.ds(base, _BPW)], idx_u)
    pltpu.sync_copy(ic_ref.at[pl.ds(base, _BPW)], idx_i)
    pltpu.sync_copy(wb_ref, wb_v)

    w = [wb_v[pl.ds(c * 16, 16)] for c in range(4)]
    b = wb_v[pl.ds(64, 16)][0]
    lane = lax.iota(jnp.int32, 16)

    def fetch(ru, ri, slot):
        pltpu.make_async_copy(
            ufT_ref.at[:, pl.ds((ru // 128) * 128, 128)],
            u_tile.at[slot], sems.at[slot]).start()
        pltpu.make_async_copy(
            ifT_ref.at[:, pl.ds((ri // 128) * 128, 128)],
            i_tile.at[slot], sems.at[slot]).start()

    def wait_slot(slot):
        pltpu.make_async_copy(
            ufT_ref.at[:, pl.ds(0, 128)], u_tile.at[slot],
            sems.at[slot]).wait()
        pltpu.make_async_copy(
            ifT_ref.at[:, pl.ds(0, 128)], i_tile.at[slot],
            sems.at[slot]).wait()

    ru0 = idx_u[pl.ds(0, 16)]
    ri0 = idx_i[pl.ds(0, 16)]
    for s in range(_NBUF):
        fetch(ru0[s], ri0[s], s)

    def group(g, carry):
        j0 = g * 16
        ru_vec = idx_u[pl.ds(j0, 16)]
        ri_vec = idx_i[pl.ds(j0, 16)]
        nxt = jnp.minimum(j0 + 16, _BPW - 16)
        ru_nxt = idx_u[pl.ds(nxt, 16)]
        ri_nxt = idx_i[pl.ds(nxt, 16)]
        last = g == _GROUPS - 1
        res = jnp.zeros((16,), jnp.float32)
        for jj in range(16):
            slot = jj % _NBUF
            ru = ru_vec[jj]
            ri = ri_vec[jj]
            wait_slot(slot)
            cuv = jnp.full((16,), ru % 128, jnp.int32)
            civ = jnp.full((16,), ri % 128, jnp.int32)
            acc = jnp.zeros((16,), jnp.float32)
            for c in range(4):
                fl = c * 16 + lane
                uv = plsc.load_gather(u_tile.at[slot], [fl, cuv])
                iv = plsc.load_gather(i_tile.at[slot], [fl, civ])
                acc = acc + uv * iv * w[c]
            if jj < 16 - _NBUF:
                run, rin = ru_vec[jj + _NBUF], ri_vec[jj + _NBUF]
            else:
                run = jnp.where(last, ru, ru_nxt[jj + _NBUF - 16])
                rin = jnp.where(last, ri, ri_nxt[jj + _NBUF - 16])
            fetch(run, rin, slot)
            s = jnp.sum(acc) + b
            res = jnp.where(lane == jj, jnp.maximum(s, 0.0), res)
        out_v[pl.ds(j0, 16)] = res
        return carry

    lax.fori_loop(0, _GROUPS, group, 0)

    for s in range(_NBUF):
        wait_slot(s)

    pltpu.sync_copy(out_v, out_ref.at[pl.ds(base, _BPW)])


def _tc_body(ucr, icr, *refs):
    blks = refs[:2 * _RPS]
    w_ref, b_ref, out_ref = refs[2 * _RPS:]
    g = pl.program_id(0)
    si = lax.broadcasted_iota(jnp.int32, (128, 1), 0)
    pos3 = (lax.broadcasted_iota(jnp.int32, (1, 8, 128), 1) * 128
            + lax.broadcasted_iota(jnp.int32, (1, 8, 128), 2))
    res = out_ref[...]
    for r in range(_RPS):
        cu = ucr[g * _RPS + r] % 128
        ci = icr[g * _RPS + r] % 128
        colu = jnp.dot(blks[r][...], (si == cu).astype(jnp.float32))
        coli = jnp.dot(blks[_RPS + r][...], (si == ci).astype(jnp.float32))
        s = jnp.maximum(jnp.sum(colu * coli * w_ref[...]) + b_ref[0, 0], 0.0)
        res = jnp.where(pos3 == (g % 128) * _RPS + r, s, res)
    out_ref[...] = res


def _tc_half(uc, ic, ufT, ifT, w2, b2):
    nsteps = _TCN // _RPS
    u_specs = [
        pl.BlockSpec((_F, 128),
                     lambda g, ucr, icr, r=r: (0, ucr[g * _RPS + r] // 128))
        for r in range(_RPS)
    ]
    i_specs = [
        pl.BlockSpec((_F, 128),
                     lambda g, ucr, icr, r=r: (0, icr[g * _RPS + r] // 128))
        for r in range(_RPS)
    ]
    grid_spec = pltpu.PrefetchScalarGridSpec(
        num_scalar_prefetch=2,
        grid=(nsteps,),
        in_specs=u_specs + i_specs + [
            pl.BlockSpec((_F, 1), lambda g, ucr, icr: (0, 0)),
            pl.BlockSpec((1, 1), lambda g, ucr, icr: (0, 0)),
        ],
        out_specs=pl.BlockSpec((1, 8, 128),
                               lambda g, ucr, icr: (g // 128, 0, 0)),
    )
    out = pl.pallas_call(
        _tc_body,
        grid_spec=grid_spec,
        out_shape=jax.ShapeDtypeStruct((_TCN // 1024, 8, 128), jnp.float32),
    )(uc, ic, *([ufT] * _RPS), *([ifT] * _RPS), w2, b2)
    return out.reshape(_TCN)


@jax.jit
def kernel(user_coordinates, item_coordinates, USER_factors, ITEM_factors,
           W, b):
    uc = user_coordinates.astype(jnp.int32)
    ic = item_coordinates.astype(jnp.int32)
    ufT = USER_factors.T
    ifT = ITEM_factors.T
    wb = jnp.concatenate([W.reshape(_F), b.reshape(1),
                          jnp.zeros((15,), jnp.float32)])

    mesh = plsc.VectorSubcoreMesh(core_axis_name="c", subcore_axis_name="s")
    run = pl.kernel(
        _body,
        mesh=mesh,
        compiler_params=pltpu.CompilerParams(needs_layout_passes=False),
        out_type=jax.ShapeDtypeStruct((_SPLIT,), jnp.float32),
        scratch_types=[
            pltpu.VMEM((_BPW,), jnp.int32),
            pltpu.VMEM((_BPW,), jnp.int32),
            pltpu.VMEM((_NBUF, _F, 128), jnp.float32),
            pltpu.VMEM((_NBUF, _F, 128), jnp.float32),
            pltpu.VMEM((80,), jnp.float32),
            pltpu.VMEM((_BPW,), jnp.float32),
            pltpu.SemaphoreType.DMA((_NBUF,)),
        ],
    )
    sc_out = run(uc[:_SPLIT], ic[:_SPLIT], ufT, ifT, wb)
    tc_out = _tc_half(uc[_SPLIT:], ic[_SPLIT:], ufT, ifT,
                      W.reshape(_F, 1), b.reshape(1, 1))
    return jnp.concatenate([sc_out, tc_out]).reshape(_B, 1)
